# baseline (speedup 1.0000x reference)
import functools

import jax
import jax.numpy as jnp
from jax import lax
from jax.experimental import pallas as pl
from jax.experimental.pallas import tpu as pltpu
from jax.experimental.pallas import tpu_sc as plsc

L = 16


@functools.cache
def _build(B, N, NC, NS):
    NW = NC * NS
    assert B % NW == 0, (B, NW)
    rows_per_w = B // NW

    CHUNK = 64
    while rows_per_w % CHUNK:
        CHUNK //= 2
    n_chunks = rows_per_w // CHUNK
    CE = CHUNK * N

    full = N // L
    offs = [i * L for i in range(full)]
    cov = full * L
    tail = cov < N
    if tail:
        offs.append(N - L)
        tail_keep = L - (N - cov)

    mesh = plsc.VectorSubcoreMesh(core_axis_name="c", subcore_axis_name="s")

    @functools.partial(
        pl.kernel,
        out_type=jax.ShapeDtypeStruct((B * N,), jnp.float32),
        mesh=mesh,
        scratch_types=[
            pltpu.VMEM((CE,), jnp.int32),
            pltpu.VMEM((CE,), jnp.float32),
            pltpu.VMEM((CE,), jnp.float32),
            pltpu.VMEM((L,), jnp.float32),
        ],
    )
    def knl(idx_hbm, chg_hbm, w_hbm, out_hbm, idx_v, chg_v, out_v, w_v):
        cid = lax.axis_index("c")
        sid = lax.axis_index("s")
        wid = sid * NC + cid
        base = wid * (rows_per_w * N)

        pltpu.sync_copy(w_hbm, w_v)
        wtab = w_v[...]
        lane = lax.iota(jnp.int32, L)
        if tail:
            keep = lane >= tail_keep
        shuf_perms = [(lane + sh) % L for sh in (8, 4, 2, 1)]

        def hsum(x):
            for perm in shuf_perms:
                x = x + x.at[perm].get(mode="promise_in_bounds")
            return x

        def chunk_body(t, carry):
            off = base + t * CE
            pltpu.sync_copy(idx_hbm.at[pl.ds(off, CE)], idx_v)
            pltpu.sync_copy(chg_hbm.at[pl.ds(off, CE)], chg_v)

            def row_body(r, carry2):
                rb = r * N
                csum = jnp.zeros((L,), jnp.float32)
                wsum = jnp.zeros((L,), jnp.float32)
                cvs = []
                wvs = []
                for o in offs:
                    s = pl.multiple_of(rb + o, 8)
                    cv = chg_v[pl.ds(s, L)]
                    iv = idx_v[pl.ds(s, L)]
                    wv = wtab.at[iv].get(mode="promise_in_bounds")
                    if tail and o == offs[-1]:
                        csum = csum + jnp.where(keep, cv, 0.0)
                        wsum = wsum + jnp.where(keep, wv, 0.0)
                    else:
                        csum = csum + cv
                        wsum = wsum + wv
                    cvs.append(cv)
                    wvs.append(wv)
                scale = (0.0 - hsum(csum)) / hsum(wsum)
                for o, cv, wv in zip(offs, cvs, wvs):
                    s = pl.multiple_of(rb + o, 8)
                    out_v[pl.ds(s, L)] = cv + scale * wv
                return carry2

            lax.fori_loop(0, CHUNK, row_body, 0)
            pltpu.sync_copy(out_v, out_hbm.at[pl.ds(off, CE)])
            return carry

        lax.fori_loop(0, n_chunks, chunk_body, 0)

    return knl


def kernel(element_idxs, raw_charges, weights):
    B, N = raw_charges.shape
    info = plsc.get_sparse_core_info()
    knl = _build(B, N, info.num_cores, info.num_subcores)
    w16 = jnp.zeros((L,), weights.dtype).at[: weights.shape[0]].set(weights)
    out = knl(
        element_idxs.reshape(B * N),
        raw_charges.reshape(B * N),
        w16,
    )
    return out.reshape(B, N)


# speedup vs baseline: 1.1069x; 1.1069x over previous
import functools

import jax
import jax.numpy as jnp
from jax import lax
from jax.experimental import pallas as pl
from jax.experimental.pallas import tpu as pltpu
from jax.experimental.pallas import tpu_sc as plsc

L = 16
RU = 4


@functools.cache
def _build(B, N, NC, NS):
    NW = NC * NS
    assert B % NW == 0, (B, NW)
    rows_per_w = B // NW

    CHUNK = 64
    while rows_per_w % CHUNK or CHUNK % RU:
        CHUNK //= 2
    n_chunks = rows_per_w // CHUNK
    CE = CHUNK * N

    full = N // L
    offs = [i * L for i in range(full)]
    cov = full * L
    tail = cov < N
    if tail:
        offs.append(N - L)
        tail_keep = L - (N - cov)

    mesh = plsc.VectorSubcoreMesh(core_axis_name="c", subcore_axis_name="s")

    @functools.partial(
        pl.kernel,
        out_type=jax.ShapeDtypeStruct((B * N,), jnp.float32),
        mesh=mesh,
        scratch_types=[
            pltpu.VMEM((CE,), jnp.int32),
            pltpu.VMEM((CE,), jnp.int32),
            pltpu.VMEM((CE,), jnp.float32),
            pltpu.VMEM((CE,), jnp.float32),
            pltpu.VMEM((CE,), jnp.float32),
            pltpu.VMEM((CE,), jnp.float32),
            pltpu.VMEM((L,), jnp.float32),
            pltpu.SemaphoreType.DMA,
            pltpu.SemaphoreType.DMA,
            pltpu.SemaphoreType.DMA,
            pltpu.SemaphoreType.DMA,
        ],
    )
    def knl(idx_hbm, chg_hbm, w_hbm, out_hbm,
            idx_v0, idx_v1, chg_v0, chg_v1, out_v0, out_v1, w_v,
            isem0, isem1, osem0, osem1):
        cid = lax.axis_index("c")
        sid = lax.axis_index("s")
        wid = sid * NC + cid
        base = wid * (rows_per_w * N)

        bufs = [
            (idx_v0, chg_v0, out_v0, isem0, osem0),
            (idx_v1, chg_v1, out_v1, isem1, osem1),
        ]

        pltpu.sync_copy(w_hbm, w_v)
        wtab = w_v[...]
        lane = lax.iota(jnp.int32, L)
        if tail:
            keep = lane >= tail_keep
        shuf_perms = [(lane + sh) % L for sh in (8, 4, 2, 1)]

        def hsum(x):
            for perm in shuf_perms:
                x = x + x.at[perm].get(mode="promise_in_bounds")
            return x

        def gather_w(iv):
            return wtab.at[iv].get(mode="promise_in_bounds")

        def row_compute(idx_v, chg_v, out_v, r):
            rb = r * N
            zero = jnp.zeros((L,), jnp.float32)
            caccs = [zero] * 4
            waccs = [zero] * 4
            for j, o in enumerate(offs):
                s = pl.multiple_of(rb + o, 8)
                cv = chg_v[pl.ds(s, L)]
                iv = idx_v[pl.ds(s, L)]
                wv = gather_w(iv)
                if tail and j == len(offs) - 1:
                    cv = jnp.where(keep, cv, 0.0)
                    wv = jnp.where(keep, wv, 0.0)
                caccs[j % 4] = caccs[j % 4] + cv
                waccs[j % 4] = waccs[j % 4] + wv
            csum = (caccs[0] + caccs[1]) + (caccs[2] + caccs[3])
            wsum = (waccs[0] + waccs[1]) + (waccs[2] + waccs[3])
            scale = (0.0 - hsum(csum)) / hsum(wsum)
            for o in offs:
                s = pl.multiple_of(rb + o, 8)
                cv = chg_v[pl.ds(s, L)]
                iv = idx_v[pl.ds(s, L)]
                out_v[pl.ds(s, L)] = cv + scale * gather_w(iv)

        def compute_chunk(idx_v, chg_v, out_v):
            def iter_body(g, carry):
                r0 = g * RU
                for u in range(RU):
                    row_compute(idx_v, chg_v, out_v, r0 + u)
                return carry

            lax.fori_loop(0, CHUNK // RU, iter_body, 0)

        def start_in(t):
            idx_v, chg_v, _, isem, _ = bufs[t % 2]
            off = base + t * CE
            pltpu.async_copy(idx_hbm.at[pl.ds(off, CE)], idx_v, isem)
            pltpu.async_copy(chg_hbm.at[pl.ds(off, CE)], chg_v, isem)

        start_in(0)
        for t in range(n_chunks):
            idx_v, chg_v, out_v, isem, osem = bufs[t % 2]
            off = base + t * CE
            pltpu.make_async_copy(idx_hbm.at[pl.ds(off, CE)], idx_v, isem).wait()
            pltpu.make_async_copy(chg_hbm.at[pl.ds(off, CE)], chg_v, isem).wait()
            if t + 1 < n_chunks:
                start_in(t + 1)
            if t >= 2:
                prev_off = base + (t - 2) * CE
                pltpu.make_async_copy(
                    out_v, out_hbm.at[pl.ds(prev_off, CE)], osem
                ).wait()
            compute_chunk(idx_v, chg_v, out_v)
            pltpu.async_copy(out_v, out_hbm.at[pl.ds(off, CE)], osem)
        for t in range(max(n_chunks - 2, 0), n_chunks):
            _, _, out_v, _, osem = bufs[t % 2]
            off = base + t * CE
            pltpu.make_async_copy(
                out_v, out_hbm.at[pl.ds(off, CE)], osem
            ).wait()

    return knl


def kernel(element_idxs, raw_charges, weights):
    B, N = raw_charges.shape
    info = plsc.get_sparse_core_info()
    knl = _build(B, N, info.num_cores, info.num_subcores)
    w16 = jnp.zeros((L,), weights.dtype).at[: weights.shape[0]].set(weights)
    out = knl(
        element_idxs.reshape(B * N),
        raw_charges.reshape(B * N),
        w16,
    )
    return out.reshape(B, N)
